# baseline (speedup 1.0000x reference)
import functools

import jax
import jax.numpy as jnp
from jax import lax
from jax.experimental import pallas as pl
from jax.experimental.pallas import tpu as pltpu
from jax.experimental.pallas import tpu_sc as plsc

_C, _T, _H, _W = 3, 64, 512, 512
_ALPHA = 4
_TS = _T // _ALPHA
_FRAME = _H * _W
_NW = 32
_CHUNK = 65536
_CHUNKS_PER_FRAME = _FRAME // _CHUNK
_TOTAL_CHUNKS = _C * _TS * _CHUNKS_PER_FRAME
_CHUNKS_PER_WORKER = _TOTAL_CHUNKS // _NW


def _fast_copy_body(x_ref, o_ref):
    o_ref[...] = x_ref[...]


_fast_copy = pl.pallas_call(
    _fast_copy_body,
    grid=(_C, _T // 8),
    in_specs=[pl.BlockSpec((1, 8, _H, _W), lambda c, i: (c, i, 0, 0))],
    out_specs=pl.BlockSpec((1, 8, _H, _W), lambda c, i: (c, i, 0, 0)),
    out_shape=jax.ShapeDtypeStruct((_C, _T, _H, _W), jnp.float32),
)


@functools.partial(
    pl.kernel,
    mesh=plsc.VectorSubcoreMesh(core_axis_name="c", subcore_axis_name="s"),
    out_type=jax.ShapeDtypeStruct((_C * _TS * _FRAME,), jnp.float32),
    scratch_types=[pltpu.VMEM((_CHUNK,), jnp.float32)],
)
def _slow_gather(frames_hbm, out_hbm, buf):
    wid = lax.axis_index("s") * 2 + lax.axis_index("c")

    def body(i, carry):
        g = wid * _CHUNKS_PER_WORKER + i
        j = g // _CHUNKS_PER_FRAME
        k = g % _CHUNKS_PER_FRAME
        c = j // _TS
        p = j % _TS
        t = (21 * p) // 5
        src = (c * _T + t) * _FRAME + k * _CHUNK
        dst = g * _CHUNK
        pltpu.sync_copy(frames_hbm.at[pl.ds(src, _CHUNK)], buf)
        pltpu.sync_copy(buf, out_hbm.at[pl.ds(dst, _CHUNK)])
        return carry

    lax.fori_loop(0, _CHUNKS_PER_WORKER, body, 0)


def kernel(frames):
    slow_flat = _slow_gather(frames.reshape(-1))
    slow = slow_flat.reshape(_C, _TS, _H, _W)
    fast = _fast_copy(frames)
    return (slow, fast)
